# baseline (speedup 1.0000x reference)
import functools

import jax
import jax.numpy as jnp
from jax import lax
from jax.experimental import pallas as pl
from jax.experimental.pallas import tpu as pltpu
from jax.experimental.pallas import tpu_sc as plsc

VOCAB = 100000
HIDDEN = 128
MAXLEN = 200
BATCH = 1024

NTOK = BATCH * MAXLEN
NW = 32
TOK_PER_W = NTOK // NW
C = 128
NCHUNK = TOK_PER_W // C
NLANE = 16
NBUF = 5
NSTEP = NCHUNK + 2
NITER = -(-NSTEP // NBUF)

_mesh = plsc.VectorSubcoreMesh(core_axis_name="c", subcore_axis_name="s")


@functools.partial(
    pl.kernel,
    mesh=_mesh,
    out_type=jax.ShapeDtypeStruct((NTOK, HIDDEN), jnp.float32),
    scratch_types=[
        pltpu.VMEM((TOK_PER_W,), jnp.int32),
        pltpu.VMEM((TOK_PER_W,), jnp.int32),
        pltpu.VMEM((MAXLEN, HIDDEN), jnp.float32),
        pltpu.VMEM((2, HIDDEN), jnp.float32),
    ]
    + [pltpu.VMEM((C, HIDDEN), jnp.float32) for _ in range(NBUF)]
    + [pltpu.VMEM_SHARED((2 * MAXLEN, HIDDEN), jnp.float32)]
    + [pltpu.SemaphoreType.DMA for _ in range(3 * NBUF)],
)
def _emb(tok_tab, seg_tab, pos_w, tok_idx, seg_idx, out, tidx_v, aidx_v, tmp_v, segv, *scr):
    rows = scr[:NBUF]
    addtab_sh = scr[NBUF]
    sem_g = scr[NBUF + 1 : 2 * NBUF + 1]
    sem_a = scr[2 * NBUF + 1 : 3 * NBUF + 1]
    sem_s = scr[3 * NBUF + 1 :]
    wid = lax.axis_index("s") * 2 + lax.axis_index("c")
    tok0 = wid * TOK_PER_W

    pltpu.sync_copy(tok_idx.at[pl.ds(tok0, TOK_PER_W)], tidx_v)
    pltpu.sync_copy(seg_idx.at[pl.ds(tok0, TOK_PER_W)], aidx_v)

    def pidx_body(v, carry):
        sl = pl.ds(v * NLANE, NLANE)
        g = tok0 + v * NLANE + lax.iota(jnp.int32, NLANE)
        aidx_v[sl] = aidx_v[sl] * MAXLEN + lax.rem(g, MAXLEN)
        return carry

    lax.fori_loop(0, TOK_PER_W // NLANE, pidx_body, 0)

    @pl.when(lax.axis_index("s") == 0)
    def _():
        pltpu.sync_copy(pos_w, tmp_v)
        pltpu.sync_copy(seg_tab, segv)

        def seg0_body(j, carry):
            for d in range(HIDDEN // NLANE):
                sl = pl.ds(d * NLANE, NLANE)
                tmp_v[j, sl] = tmp_v[j, sl] + segv[0, sl]
            return carry

        lax.fori_loop(0, MAXLEN, seg0_body, 0)
        pltpu.sync_copy(tmp_v, addtab_sh.at[pl.ds(0, MAXLEN)])

        def seg1_body(j, carry):
            for d in range(HIDDEN // NLANE):
                sl = pl.ds(d * NLANE, NLANE)
                tmp_v[j, sl] = tmp_v[j, sl] + (segv[1, sl] - segv[0, sl])
            return carry

        lax.fori_loop(0, MAXLEN, seg1_body, 0)
        pltpu.sync_copy(tmp_v, addtab_sh.at[pl.ds(MAXLEN, MAXLEN)])

    plsc.subcore_barrier()

    def step_body(it, carry):
        for b in range(NBUF):
            s = it * NBUF + b
            bg, ba, bs = b, (b - 1) % NBUF, (b - 2) % NBUF

            @pl.when(jnp.logical_and(s >= NBUF, s < NCHUNK))
            def _():
                pltpu.make_async_copy(
                    rows[bg], out.at[pl.ds(0, C)], sem_s[bg]
                ).wait()

            @pl.when(s < NCHUNK)
            def _():
                pltpu.async_copy(
                    tok_tab.at[tidx_v.at[pl.ds(s * C, C)]], rows[bg], sem_g[bg]
                )

            @pl.when(jnp.logical_and(s >= 1, s <= NCHUNK))
            def _():
                c1 = s - 1
                pltpu.make_async_copy(
                    tok_tab.at[tidx_v.at[pl.ds(0, C)]], rows[ba], sem_g[ba]
                ).wait()
                pltpu.async_copy(
                    addtab_sh.at[aidx_v.at[pl.ds(c1 * C, C)]],
                    rows[ba],
                    sem_a[ba],
                    add=True,
                )

            @pl.when(jnp.logical_and(s >= 2, s <= NCHUNK + 1))
            def _():
                c2 = s - 2
                pltpu.make_async_copy(
                    addtab_sh.at[aidx_v.at[pl.ds(0, C)]], rows[bs], sem_a[bs]
                ).wait()
                pltpu.async_copy(rows[bs], out.at[pl.ds(tok0 + c2 * C, C)], sem_s[bs])

        return carry

    lax.fori_loop(0, NITER, step_body, 0)

    for c in range(NCHUNK - NBUF, NCHUNK):
        pltpu.make_async_copy(
            rows[c % NBUF], out.at[pl.ds(0, C)], sem_s[c % NBUF]
        ).wait()


def kernel(tokens, segments, token_table, segment_table, pos_w):
    tok_idx = tokens.astype(jnp.int32).reshape(NTOK)
    seg_idx = segments.astype(jnp.int32).reshape(NTOK)
    out = _emb(token_table, segment_table, pos_w, tok_idx, seg_idx)
    return out.reshape(BATCH, MAXLEN, HIDDEN)


# speedup vs baseline: 1.1661x; 1.1661x over previous
import functools

import jax
import jax.numpy as jnp
from jax import lax
from jax.experimental import pallas as pl
from jax.experimental.pallas import tpu as pltpu
from jax.experimental.pallas import tpu_sc as plsc

VOCAB = 100000
HIDDEN = 128
MAXLEN = 200
BATCH = 1024

NTOK = BATCH * MAXLEN
NW = 32
TOK_PER_W = NTOK // NW
C = 128
NCHUNK = TOK_PER_W // C
NLANE = 16
NBUF = 5
NSTEP = NCHUNK + 2
NITER = -(-NSTEP // NBUF)


def _prep_body(seg_ref, segtab_ref, pos_ref, cidx_ref, addtab_ref):
    pos_ids = lax.broadcasted_iota(jnp.int32, (BATCH, MAXLEN), 1)
    cidx_ref[...] = seg_ref[...].astype(jnp.int32) * MAXLEN + pos_ids
    addtab_ref[0:MAXLEN, :] = pos_ref[...] + segtab_ref[0:1, :]
    addtab_ref[MAXLEN : 2 * MAXLEN, :] = pos_ref[...] + segtab_ref[1:2, :]


def _preprocess(segments, segment_table, pos_w):
    return pl.pallas_call(
        _prep_body,
        out_shape=(
            jax.ShapeDtypeStruct((BATCH, MAXLEN), jnp.int32),
            jax.ShapeDtypeStruct((2 * MAXLEN, HIDDEN), jnp.float32),
        ),
    )(segments, segment_table, pos_w)


_mesh = plsc.VectorSubcoreMesh(core_axis_name="c", subcore_axis_name="s")


@functools.partial(
    pl.kernel,
    mesh=_mesh,
    out_type=jax.ShapeDtypeStruct((NTOK, HIDDEN), jnp.float32),
    scratch_types=[
        pltpu.VMEM((TOK_PER_W,), jnp.int32),
        pltpu.VMEM((TOK_PER_W,), jnp.int32),
    ]
    + [pltpu.VMEM((C, HIDDEN), jnp.float32) for _ in range(NBUF)]
    + [pltpu.VMEM_SHARED((2 * MAXLEN, HIDDEN), jnp.float32)]
    + [pltpu.SemaphoreType.DMA for _ in range(3 * NBUF)],
)
def _emb(tok_tab, add_tab, tok_idx, cidx, out, tidx_v, aidx_v, *scr):
    rows = scr[:NBUF]
    addtab_sh = scr[NBUF]
    sem_g = scr[NBUF + 1 : 2 * NBUF + 1]
    sem_a = scr[2 * NBUF + 1 : 3 * NBUF + 1]
    sem_s = scr[3 * NBUF + 1 :]
    wid = lax.axis_index("s") * 2 + lax.axis_index("c")
    tok0 = wid * TOK_PER_W

    pltpu.sync_copy(tok_idx.at[pl.ds(tok0, TOK_PER_W)], tidx_v)
    pltpu.sync_copy(cidx.at[pl.ds(tok0, TOK_PER_W)], aidx_v)

    @pl.when(lax.axis_index("s") == 0)
    def _():
        pltpu.sync_copy(add_tab, addtab_sh)

    plsc.subcore_barrier()

    def step_body(it, carry):
        for b in range(NBUF):
            s = it * NBUF + b
            bg, ba, bs = b, (b - 1) % NBUF, (b - 2) % NBUF

            @pl.when(jnp.logical_and(s >= NBUF, s < NCHUNK))
            def _():
                pltpu.make_async_copy(
                    rows[bg], out.at[pl.ds(0, C)], sem_s[bg]
                ).wait()

            @pl.when(s < NCHUNK)
            def _():
                pltpu.async_copy(
                    tok_tab.at[tidx_v.at[pl.ds(s * C, C)]], rows[bg], sem_g[bg]
                )

            @pl.when(jnp.logical_and(s >= 1, s <= NCHUNK))
            def _():
                c1 = s - 1
                pltpu.make_async_copy(
                    tok_tab.at[tidx_v.at[pl.ds(0, C)]], rows[ba], sem_g[ba]
                ).wait()
                pltpu.async_copy(
                    addtab_sh.at[aidx_v.at[pl.ds(c1 * C, C)]],
                    rows[ba],
                    sem_a[ba],
                    add=True,
                )

            @pl.when(jnp.logical_and(s >= 2, s <= NCHUNK + 1))
            def _():
                c2 = s - 2
                pltpu.make_async_copy(
                    addtab_sh.at[aidx_v.at[pl.ds(0, C)]], rows[bs], sem_a[bs]
                ).wait()
                pltpu.async_copy(rows[bs], out.at[pl.ds(tok0 + c2 * C, C)], sem_s[bs])

        return carry

    lax.fori_loop(0, NITER, step_body, 0)

    for c in range(NCHUNK - NBUF, NCHUNK):
        pltpu.make_async_copy(
            rows[c % NBUF], out.at[pl.ds(0, C)], sem_s[c % NBUF]
        ).wait()


def kernel(tokens, segments, token_table, segment_table, pos_w):
    cidx2, addtab = _preprocess(segments, segment_table, pos_w)
    tok_idx = tokens.astype(jnp.int32).reshape(NTOK)
    cidx = cidx2.reshape(NTOK)
    out = _emb(token_table, addtab, tok_idx, cidx)
    return out.reshape(BATCH, MAXLEN, HIDDEN)


# speedup vs baseline: 1.1661x; 1.0000x over previous
import functools

import jax
import jax.numpy as jnp
from jax import lax
from jax.experimental import pallas as pl
from jax.experimental.pallas import tpu as pltpu
from jax.experimental.pallas import tpu_sc as plsc

VOCAB = 100000
HIDDEN = 128
MAXLEN = 200
BATCH = 1024

NTOK = BATCH * MAXLEN
NW = 32
TOK_PER_W = NTOK // NW
C = 128
NCHUNK = TOK_PER_W // C
NLANE = 16
NBUF = 6
NSTEP = NCHUNK + 3
NITER = -(-NSTEP // NBUF)


def _prep_body(seg_ref, segtab_ref, pos_ref, cidx_ref, addtab_ref):
    pos_ids = lax.broadcasted_iota(jnp.int32, (BATCH, MAXLEN), 1)
    cidx_ref[...] = seg_ref[...].astype(jnp.int32) * MAXLEN + pos_ids
    addtab_ref[0:MAXLEN, :] = pos_ref[...] + segtab_ref[0:1, :]
    addtab_ref[MAXLEN : 2 * MAXLEN, :] = pos_ref[...] + segtab_ref[1:2, :]


def _preprocess(segments, segment_table, pos_w):
    return pl.pallas_call(
        _prep_body,
        out_shape=(
            jax.ShapeDtypeStruct((BATCH, MAXLEN), jnp.int32),
            jax.ShapeDtypeStruct((2 * MAXLEN, HIDDEN), jnp.float32),
        ),
    )(segments, segment_table, pos_w)


_mesh = plsc.VectorSubcoreMesh(core_axis_name="c", subcore_axis_name="s")


@functools.partial(
    pl.kernel,
    mesh=_mesh,
    out_type=jax.ShapeDtypeStruct((NTOK, HIDDEN), jnp.float32),
    scratch_types=[
        pltpu.VMEM((TOK_PER_W,), jnp.int32),
        pltpu.VMEM((TOK_PER_W,), jnp.int32),
    ]
    + [pltpu.VMEM((C, HIDDEN), jnp.float32) for _ in range(NBUF)]
    + [pltpu.VMEM_SHARED((2 * MAXLEN, HIDDEN), jnp.float32)]
    + [pltpu.SemaphoreType.DMA for _ in range(3 * NBUF)],
)
def _emb(tok_tab, add_tab, tok_idx, cidx, out, tidx_v, aidx_v, *scr):
    rows = scr[:NBUF]
    addtab_sh = scr[NBUF]
    sem_g = scr[NBUF + 1 : 2 * NBUF + 1]
    sem_a = scr[2 * NBUF + 1 : 3 * NBUF + 1]
    sem_s = scr[3 * NBUF + 1 :]
    wid = lax.axis_index("s") * 2 + lax.axis_index("c")
    tok0 = wid * TOK_PER_W

    pltpu.sync_copy(tok_idx.at[pl.ds(tok0, TOK_PER_W)], tidx_v)
    pltpu.sync_copy(cidx.at[pl.ds(tok0, TOK_PER_W)], aidx_v)

    @pl.when(lax.axis_index("s") == 0)
    def _():
        pltpu.sync_copy(add_tab, addtab_sh)

    plsc.subcore_barrier()

    def step_body(it, carry):
        for b in range(NBUF):
            s = it * NBUF + b
            bg, ba, bs = b, (b - 2) % NBUF, (b - 3) % NBUF

            @pl.when(jnp.logical_and(s >= NBUF, s < NCHUNK))
            def _():
                pltpu.make_async_copy(
                    rows[bg], out.at[pl.ds(0, C)], sem_s[bg]
                ).wait()

            @pl.when(s < NCHUNK)
            def _():
                pltpu.async_copy(
                    tok_tab.at[tidx_v.at[pl.ds(s * C, C)]], rows[bg], sem_g[bg]
                )

            @pl.when(jnp.logical_and(s >= 2, s <= NCHUNK + 1))
            def _():
                c1 = s - 2
                pltpu.make_async_copy(
                    tok_tab.at[tidx_v.at[pl.ds(0, C)]], rows[ba], sem_g[ba]
                ).wait()
                pltpu.async_copy(
                    addtab_sh.at[aidx_v.at[pl.ds(c1 * C, C)]],
                    rows[ba],
                    sem_a[ba],
                    add=True,
                )

            @pl.when(jnp.logical_and(s >= 3, s <= NCHUNK + 2))
            def _():
                c2 = s - 3
                pltpu.make_async_copy(
                    addtab_sh.at[aidx_v.at[pl.ds(0, C)]], rows[bs], sem_a[bs]
                ).wait()
                pltpu.async_copy(rows[bs], out.at[pl.ds(tok0 + c2 * C, C)], sem_s[bs])

        return carry

    lax.fori_loop(0, NITER, step_body, 0)

    for c in range(NCHUNK - NBUF, NCHUNK):
        pltpu.make_async_copy(
            rows[c % NBUF], out.at[pl.ds(0, C)], sem_s[c % NBUF]
        ).wait()


def kernel(tokens, segments, token_table, segment_table, pos_w):
    cidx2, addtab = _preprocess(segments, segment_table, pos_w)
    tok_idx = tokens.astype(jnp.int32).reshape(NTOK)
    cidx = cidx2.reshape(NTOK)
    out = _emb(token_table, addtab, tok_idx, cidx)
    return out.reshape(BATCH, MAXLEN, HIDDEN)


# speedup vs baseline: 1.1789x; 1.0109x over previous
import functools

import jax
import jax.numpy as jnp
from jax import lax
from jax.experimental import pallas as pl
from jax.experimental.pallas import tpu as pltpu
from jax.experimental.pallas import tpu_sc as plsc

VOCAB = 100000
HIDDEN = 128
MAXLEN = 200
BATCH = 1024

NTOK = BATCH * MAXLEN
NW = 32
TOK_PER_W = NTOK // NW
C = 128
NCHUNK = TOK_PER_W // C
NLANE = 16
NBUF = 5
NSTEP = NCHUNK + 2
NITER = -(-NSTEP // NBUF)


def _addtab_body(seg_ref, pos_ref, out_ref):
    out_ref[0:MAXLEN, :] = pos_ref[...] + seg_ref[0:1, :]
    out_ref[MAXLEN : 2 * MAXLEN, :] = pos_ref[...] + seg_ref[1:2, :]


def _build_addtab(segment_table, pos_w):
    return pl.pallas_call(
        _addtab_body,
        out_shape=jax.ShapeDtypeStruct((2 * MAXLEN, HIDDEN), jnp.float32),
    )(segment_table, pos_w)


_mesh = plsc.VectorSubcoreMesh(core_axis_name="c", subcore_axis_name="s")


@functools.partial(
    pl.kernel,
    mesh=_mesh,
    out_type=jax.ShapeDtypeStruct((NTOK, HIDDEN), jnp.float32),
    scratch_types=[
        pltpu.VMEM((TOK_PER_W,), jnp.int32),
        pltpu.VMEM((TOK_PER_W,), jnp.int32),
    ]
    + [pltpu.VMEM((C, HIDDEN), jnp.float32) for _ in range(NBUF)]
    + [pltpu.VMEM_SHARED((2 * MAXLEN, HIDDEN), jnp.float32)]
    + [pltpu.SemaphoreType.DMA for _ in range(3 * NBUF)],
)
def _emb(tok_tab, add_tab, tok_idx, seg_idx, out, tidx_v, aidx_v, *scr):
    rows = scr[:NBUF]
    addtab_sh = scr[NBUF]
    sem_g = scr[NBUF + 1 : 2 * NBUF + 1]
    sem_a = scr[2 * NBUF + 1 : 3 * NBUF + 1]
    sem_s = scr[3 * NBUF + 1 :]
    wid = lax.axis_index("s") * 2 + lax.axis_index("c")
    tok0 = wid * TOK_PER_W

    pltpu.sync_copy(tok_idx.at[pl.ds(tok0, TOK_PER_W)], tidx_v)
    pltpu.sync_copy(seg_idx.at[pl.ds(tok0, TOK_PER_W)], aidx_v)

    def pidx_body(v, carry):
        sl = pl.ds(v * NLANE, NLANE)
        g = v * NLANE + lax.iota(jnp.int32, NLANE)
        aidx_v[sl] = aidx_v[sl] * MAXLEN + lax.rem(g, MAXLEN)
        return carry

    lax.fori_loop(0, TOK_PER_W // NLANE, pidx_body, 0)

    @pl.when(lax.axis_index("s") == 0)
    def _():
        pltpu.sync_copy(add_tab, addtab_sh)

    plsc.subcore_barrier()

    def step_body(it, carry):
        for b in range(NBUF):
            s = it * NBUF + b
            bg, ba, bs = b, (b - 1) % NBUF, (b - 2) % NBUF

            @pl.when(jnp.logical_and(s >= NBUF, s < NCHUNK))
            def _():
                pltpu.make_async_copy(
                    rows[bg], out.at[pl.ds(0, C)], sem_s[bg]
                ).wait()

            @pl.when(s < NCHUNK)
            def _():
                pltpu.async_copy(
                    tok_tab.at[tidx_v.at[pl.ds(s * C, C)]], rows[bg], sem_g[bg]
                )

            @pl.when(jnp.logical_and(s >= 1, s <= NCHUNK))
            def _():
                c1 = s - 1
                pltpu.make_async_copy(
                    tok_tab.at[tidx_v.at[pl.ds(0, C)]], rows[ba], sem_g[ba]
                ).wait()
                pltpu.async_copy(
                    addtab_sh.at[aidx_v.at[pl.ds(c1 * C, C)]],
                    rows[ba],
                    sem_a[ba],
                    add=True,
                )

            @pl.when(jnp.logical_and(s >= 2, s <= NCHUNK + 1))
            def _():
                c2 = s - 2
                pltpu.make_async_copy(
                    addtab_sh.at[aidx_v.at[pl.ds(0, C)]], rows[bs], sem_a[bs]
                ).wait()
                pltpu.async_copy(rows[bs], out.at[pl.ds(tok0 + c2 * C, C)], sem_s[bs])

        return carry

    lax.fori_loop(0, NITER, step_body, 0)

    for c in range(NCHUNK - NBUF, NCHUNK):
        pltpu.make_async_copy(
            rows[c % NBUF], out.at[pl.ds(0, C)], sem_s[c % NBUF]
        ).wait()


def kernel(tokens, segments, token_table, segment_table, pos_w):
    tok_idx = tokens.astype(jnp.int32).reshape(NTOK)
    seg_idx = segments.astype(jnp.int32).reshape(NTOK)
    addtab = _build_addtab(segment_table, pos_w)
    out = _emb(token_table, addtab, tok_idx, seg_idx)
    return out.reshape(BATCH, MAXLEN, HIDDEN)
